# baseline (speedup 1.0000x reference)
import functools

import jax
import jax.numpy as jnp
from jax import lax
from jax.experimental import pallas as pl
from jax.experimental.pallas import tpu as pltpu
from jax.experimental.pallas import tpu_sc as plsc

B = 4096
FT = 26
FS = 2
V = 100000
D = 16
HIST = 50
HIST_PAD = 64

NC, NS = 2, 16
NW = NC * NS
BPW = B // NW
CB = 8
NCHUNK = BPW // CB
PAIRS = NCHUNK // 2
SU = CB * HIST
SP = CB * HIST_PAD


def _sc_logits():
    mesh = plsc.VectorSubcoreMesh(core_axis_name="c", subcore_axis_name="s")

    @functools.partial(
        pl.kernel,
        out_type=jax.ShapeDtypeStruct((B,), jnp.float32),
        mesh=mesh,
        scratch_types=[
            pltpu.VMEM((FT, BPW), jnp.int32),
            pltpu.VMEM((FS, BPW * HIST), jnp.int32),
            pltpu.VMEM((FS, BPW * HIST_PAD), jnp.int32),
            pltpu.VMEM((FT, BPW), jnp.float32),
            pltpu.VMEM((FS, BPW * HIST_PAD), jnp.float32),
            pltpu.VMEM((2, FT, CB, D), jnp.float32),
            pltpu.VMEM((2, FS, SU, D), jnp.float32),
            pltpu.VMEM((BPW,), jnp.float32),
            pltpu.SemaphoreType.DMA,
            pltpu.SemaphoreType.DMA,
            pltpu.SemaphoreType.DMA,
        ],
        compiler_params=pltpu.CompilerParams(
            needs_layout_passes=False, use_tc_tiling_on_sc=False),
    )
    def sc_fm(tokT_h, sequ_h, seqp_h, w1t_h, w1s_h, w2t_h, w2s_h,
              out_h, tok_iv, sequ_iv, seqp_iv, w1t_v, w1s_v, tok_rows,
              seq_rows, logits_v, sem0, sem1, semw):
        wid = lax.axis_index("s") * NC + lax.axis_index("c")
        sems = (sem0, sem1)

        def fire(c, slot, sem):
            for f in range(FT):
                pltpu.async_copy(
                    w2t_h.at[f].at[tok_iv.at[f, pl.ds(c * CB, CB)]],
                    tok_rows.at[slot, f], sem)
            for f in range(FS):
                pltpu.async_copy(
                    w2s_h.at[f].at[sequ_iv.at[f, pl.ds(c * SU, SU)]],
                    seq_rows.at[slot, f], sem)

        def drain(slot, sem):
            for f in range(FT):
                pltpu.make_async_copy(
                    w2t_h.at[0].at[pl.ds(0, CB)], tok_rows.at[slot, f],
                    sem).wait()
            for f in range(FS):
                pltpu.make_async_copy(
                    w2s_h.at[0].at[pl.ds(0, SU)], seq_rows.at[slot, f],
                    sem).wait()

        pltpu.sync_copy(tokT_h.at[:, pl.ds(wid * BPW, BPW)], tok_iv)
        pltpu.sync_copy(sequ_h.at[:, pl.ds(wid * BPW * HIST, BPW * HIST)],
                        sequ_iv)
        pltpu.sync_copy(seqp_h.at[:, pl.ds(wid * BPW * HIST_PAD,
                                           BPW * HIST_PAD)], seqp_iv)
        for f in range(FT):
            pltpu.async_copy(w1t_h.at[f].at[tok_iv.at[f]], w1t_v.at[f], semw)
        for f in range(FS):
            pltpu.async_copy(w1s_h.at[f].at[seqp_iv.at[f]], w1s_v.at[f], semw)
        fire(0, 0, sem0)
        for f in range(FT):
            pltpu.make_async_copy(w1t_h.at[0].at[pl.ds(0, BPW)],
                                  w1t_v.at[f], semw).wait()
        for f in range(FS):
            pltpu.make_async_copy(w1s_h.at[0].at[pl.ds(0, BPW * HIST_PAD)],
                                  w1s_v.at[f], semw).wait()

        def pair_body(c2, carry):
            lane = lax.iota(jnp.int32, 16)
            lv = jnp.zeros((16,), jnp.float32)
            fo_pair = jnp.zeros((16,), jnp.float32)
            for f in range(FT):
                fo_pair = fo_pair + w1t_v[f, pl.ds(c2 * 16, 16)]
            for slot in (0, 1):
                c = c2 * 2 + slot
                nslot = 1 - slot

                @pl.when(c + 1 < NCHUNK)
                def _():
                    fire(c + 1, nslot, sems[nslot])

                drain(slot, sems[slot])
                for b in range(CB):
                    s = jnp.zeros((D,), jnp.float32)
                    q = jnp.zeros((D,), jnp.float32)
                    for j in range(FT):
                        r = tok_rows[slot, j, b]
                        s = s + r
                        q = q + r * r
                    fo_vec = jnp.zeros((16,), jnp.float32)
                    for f in range(FS):
                        m = jnp.zeros((D,), jnp.float32)
                        for l in range(HIST):
                            m = m + seq_rows[slot, f, b * HIST + l]
                        cnt = jnp.zeros((16,), jnp.float32)
                        sv = jnp.zeros((16,), jnp.float32)
                        for k in range(HIST_PAD // 16):
                            off = c * SP + b * HIST_PAD + k * 16
                            sl = seqp_iv[f, pl.ds(off, 16)]
                            cnt = cnt + (sl != 0).astype(jnp.float32)
                            sv = sv + w1s_v[f, pl.ds(off, 16)]
                        inv = 1.0 / jnp.maximum(
                            jnp.broadcast_to(jnp.sum(cnt), (16,)), 1.0)
                        mean = m * inv
                        s = s + mean
                        q = q + mean * mean
                        fo_vec = fo_vec + sv * inv
                    z = jnp.sum(s * s - q + fo_vec)
                    lv = lv + jnp.where(lane == slot * CB + b,
                                        jnp.broadcast_to(z, (16,)),
                                        jnp.zeros((16,), jnp.float32))
            logits_v[pl.ds(c2 * 16, 16)] = lv + fo_pair
            return carry

        lax.fori_loop(0, PAIRS, pair_body, 0)
        pltpu.sync_copy(logits_v, out_h.at[pl.ds(wid * BPW, BPW)])

    return sc_fm


def _loss_body(z_ref, y_ref, bias_ref, o_ref):
    z = z_ref[...] + bias_ref[0, 0]
    y = y_ref[...]
    l = jnp.maximum(z, 0.0) - z * y + jnp.log1p(jnp.exp(-jnp.abs(z)))
    o_ref[...] = jnp.broadcast_to(jnp.sum(l) * (1.0 / B), (1, 1))


def kernel(token_field_values, token_sequence_field_values, labels, global_bias,
           W1_token, W1_seq, W2_token, W2_seq):
    tok = token_field_values.astype(jnp.int32)
    seq = token_sequence_field_values.astype(jnp.int32)

    tokT = tok.T
    seqT = seq.transpose(1, 0, 2)
    sequ = seqT.reshape(FS, B * HIST)
    seqp = jnp.concatenate(
        [seqT, jnp.zeros((FS, B, HIST_PAD - HIST), jnp.int32)],
        axis=2).reshape(FS, B * HIST_PAD)

    w1t = W1_token[:, :, 0]
    w1s = W1_seq[:, :, 0]

    logits = _sc_logits()(tokT, sequ, seqp, w1t, w1s, W2_token, W2_seq)

    loss = pl.pallas_call(
        _loss_body,
        out_shape=jax.ShapeDtypeStruct((1, 1), jnp.float32),
    )(logits.reshape(32, 128), labels.reshape(32, 128),
      global_bias.reshape(1, 1))
    return loss.reshape(())


# speedup vs baseline: 1.2185x; 1.2185x over previous
import functools

import jax
import jax.numpy as jnp
from jax import lax
from jax.experimental import pallas as pl
from jax.experimental.pallas import tpu as pltpu
from jax.experimental.pallas import tpu_sc as plsc

B = 4096
FT = 26
FS = 2
V = 100000
D = 16
HIST = 50
FT_PAD = 32
HIST_PAD = 64

NC, NS = 2, 16
NW = NC * NS
BPW = B // NW
CB = 4
NCHUNK = BPW // CB
QUADS = NCHUNK // 4

TI = CB * FT
SI = CB * FS * HIST
TP = CB * FT_PAD
SP = CB * FS * HIST_PAD
NTI = NCHUNK * TI
NSI = NCHUNK * SI
NTP = NCHUNK * TP
NSP = NCHUNK * SP


def _sc_logits():
    mesh = plsc.VectorSubcoreMesh(core_axis_name="c", subcore_axis_name="s")

    @functools.partial(
        pl.kernel,
        out_type=jax.ShapeDtypeStruct((B,), jnp.float32),
        mesh=mesh,
        scratch_types=[
            pltpu.VMEM((NTI + 16,), jnp.int32),
            pltpu.VMEM((NTI,), jnp.int32),
            pltpu.VMEM((NSI,), jnp.int32),
            pltpu.VMEM((NTP,), jnp.int32),
            pltpu.VMEM((NSP,), jnp.int32),
            pltpu.VMEM((2, TI, 128), jnp.float32),
            pltpu.VMEM((2, SI, D), jnp.float32),
            pltpu.VMEM((2, TP), jnp.float32),
            pltpu.VMEM((2, SP), jnp.float32),
            pltpu.VMEM((BPW,), jnp.float32),
            pltpu.SemaphoreType.DMA,
            pltpu.SemaphoreType.DMA,
        ],
        compiler_params=pltpu.CompilerParams(
            needs_layout_passes=False, use_tc_tiling_on_sc=False),
    )
    def sc_fm(tok_i_h, seq_i_h, tok_p_h, seq_p_h, w1t_h, w1s_h, w2t_h, w2s_h,
              out_h, tok_iv, tok_ov, seq_iv, tok_pv, seq_pv, tok_rows,
              seq_rows, tok_w1v, seq_w1v, logits_v, sem0, sem1):
        wid = lax.axis_index("s") * NC + lax.axis_index("c")
        sems = (sem0, sem1)

        def fire(c, slot, sem):
            pltpu.async_copy(
                w2t_h.at[tok_ov.at[pl.ds(c * TI, TI)]], tok_rows.at[slot], sem)
            pltpu.async_copy(
                w2s_h.at[seq_iv.at[pl.ds(c * SI, SI)]], seq_rows.at[slot], sem)
            pltpu.async_copy(
                w1t_h.at[tok_pv.at[pl.ds(c * TP, TP)]], tok_w1v.at[slot], sem)
            pltpu.async_copy(
                w1s_h.at[seq_pv.at[pl.ds(c * SP, SP)]], seq_w1v.at[slot], sem)

        def drain(slot, sem):
            pltpu.make_async_copy(
                w2t_h.at[pl.ds(0, TI)], tok_rows.at[slot], sem).wait()
            pltpu.make_async_copy(
                w2s_h.at[pl.ds(0, SI)], seq_rows.at[slot], sem).wait()
            pltpu.make_async_copy(
                w1t_h.at[pl.ds(0, TP)], tok_w1v.at[slot], sem).wait()
            pltpu.make_async_copy(
                w1s_h.at[pl.ds(0, SP)], seq_w1v.at[slot], sem).wait()

        pltpu.sync_copy(tok_i_h.at[pl.ds(wid * NTI, NTI)],
                        tok_iv.at[pl.ds(0, NTI)])
        pltpu.sync_copy(seq_i_h.at[pl.ds(wid * NSI, NSI)], seq_iv)
        pltpu.sync_copy(tok_p_h.at[pl.ds(wid * NTP, NTP)], tok_pv)
        pltpu.sync_copy(seq_p_h.at[pl.ds(wid * NSP, NSP)], seq_pv)
        for i in range(NTI // 16):
            tok_ov[pl.ds(i * 16, 16)] = lax.shift_right_logical(
                tok_iv[pl.ds(i * 16, 16)], 3)
        fire(0, 0, sem0)

        def quad_body(c4, carry):
            lane = lax.iota(jnp.int32, 16)
            lv = jnp.zeros((16,), jnp.float32)
            for u in (0, 1, 2, 3):
                slot = u % 2
                c = c4 * 4 + u
                nslot = 1 - slot

                @pl.when(c + 1 < NCHUNK)
                def _():
                    fire(c + 1, nslot, sems[nslot])

                drain(slot, sems[slot])
                for b in range(CB):
                    s = jnp.zeros((D,), jnp.float32)
                    q = jnp.zeros((D,), jnp.float32)
                    base = c * TI + b * FT
                    va = jnp.bitwise_and(tok_iv[pl.ds(base, 16)], 7) * 16
                    vb = jnp.bitwise_and(tok_iv[pl.ds(base + 16, 16)], 7) * 16
                    for j in range(FT):
                        sub = va[j] if j < 16 else vb[j - 16]
                        r = tok_rows[slot, b * FT + j, pl.ds(sub, 16)]
                        s = s + r
                        q = q + r * r
                    fo_vec = (tok_w1v[slot, pl.ds(b * FT_PAD, 16)]
                              + tok_w1v[slot, pl.ds(b * FT_PAD + 16, 16)])
                    for f in range(FS):
                        m = jnp.zeros((D,), jnp.float32)
                        for l in range(HIST):
                            m = m + seq_rows[slot, (b * FS + f) * HIST + l]
                        cnt = jnp.zeros((16,), jnp.float32)
                        sv = jnp.zeros((16,), jnp.float32)
                        for k in range(HIST_PAD // 16):
                            off = (b * FS + f) * HIST_PAD + k * 16
                            sl = seq_pv[pl.ds(c * SP + off, 16)]
                            cnt = cnt + (sl != f * V).astype(jnp.float32)
                            sv = sv + seq_w1v[slot, pl.ds(off, 16)]
                        inv = 1.0 / jnp.maximum(
                            jnp.broadcast_to(jnp.sum(cnt), (16,)), 1.0)
                        mean = m * inv
                        s = s + mean
                        q = q + mean * mean
                        fo_vec = fo_vec + sv * inv
                    z = jnp.sum(s * s - q + fo_vec)
                    lv = lv + jnp.where(lane == u * CB + b,
                                        jnp.broadcast_to(z, (16,)),
                                        jnp.zeros((16,), jnp.float32))
            logits_v[pl.ds(c4 * 16, 16)] = lv
            return carry

        lax.fori_loop(0, QUADS, quad_body, 0)
        pltpu.sync_copy(logits_v, out_h.at[pl.ds(wid * BPW, BPW)])

    return sc_fm


def _loss_body(z_ref, y_ref, bias_ref, o_ref):
    z = z_ref[...] + bias_ref[0, 0]
    y = y_ref[...]
    l = jnp.maximum(z, 0.0) - z * y + jnp.log1p(jnp.exp(-jnp.abs(z)))
    o_ref[...] = jnp.broadcast_to(jnp.sum(l) * (1.0 / B), (1, 1))


def kernel(token_field_values, token_sequence_field_values, labels, global_bias,
           W1_token, W1_seq, W2_token, W2_seq):
    tok = token_field_values.astype(jnp.int32)
    seq = token_sequence_field_values.astype(jnp.int32)
    off_t = (jnp.arange(FT, dtype=jnp.int32) * V)[None, :]
    off_s = (jnp.arange(FS, dtype=jnp.int32) * V)[None, :, None]

    tok_i = tok + off_t
    tok_p = jnp.concatenate(
        [tok_i, jnp.zeros((B, FT_PAD - FT), jnp.int32)], axis=1)
    seq_off = seq + off_s
    seq_i = seq_off.reshape(B, FS * HIST)
    seq_p = jnp.concatenate(
        [seq_off, jnp.broadcast_to(off_s, (B, FS, HIST_PAD - HIST))],
        axis=2).reshape(B, FS * HIST_PAD)

    w1t = W1_token.reshape(FT * V)
    w1s = W1_seq.reshape(FS * V)
    w2t_oct = W2_token.reshape(FT * V // 8, 128)
    w2s = W2_seq.reshape(FS * V, D)

    logits = _sc_logits()(tok_i.reshape(-1), seq_i.reshape(-1),
                          tok_p.reshape(-1), seq_p.reshape(-1),
                          w1t, w1s, w2t_oct, w2s)

    loss = pl.pallas_call(
        _loss_body,
        out_shape=jax.ShapeDtypeStruct((1, 1), jnp.float32),
    )(logits.reshape(32, 128), labels.reshape(32, 128),
      global_bias.reshape(1, 1))
    return loss.reshape(())
